# baseline (speedup 1.0000x reference)
import jax
import jax.numpy as jnp
from jax.experimental import pallas as pl
from jax.experimental.pallas import tpu as pltpu

B, S, F = 4096, 4096, 4
R, L = 128, 128
BB = 128


def _mask_copy_kernel(idx_ref, x_ref, o_ref):
    i = pl.program_id(0)
    idx = idx_ref[0, i % 8, :].reshape(BB, 1, 1)
    row = jax.lax.broadcasted_iota(jnp.int32, (BB, R, L), 1)
    lane = jax.lax.broadcasted_iota(jnp.int32, (BB, R, L), 2)
    hit = ((row >> 2) == (idx >> 7)) & (lane == (idx & 127))
    o_ref[...] = jnp.where(hit, 0.0, x_ref[...])


def kernel(x, idx):
    b, s, f = x.shape
    v = x.reshape(b, s // L, L, f).transpose(0, 1, 3, 2).reshape(b, R, L)
    idx3 = idx.reshape(b // 1024, 8, 128)
    out = pl.pallas_call(
        _mask_copy_kernel,
        grid=(b // BB,),
        in_specs=[
            pl.BlockSpec((1, 8, 128), lambda i: (i // 8, 0, 0)),
            pl.BlockSpec((BB, R, L), lambda i: (i, 0, 0)),
        ],
        out_specs=pl.BlockSpec((BB, R, L), lambda i: (i, 0, 0)),
        out_shape=jax.ShapeDtypeStruct((b, R, L), x.dtype),
        compiler_params=pltpu.CompilerParams(vmem_limit_bytes=60000 * 1024),
    )(idx3, v)
    return out.reshape(b, s // L, f, L).transpose(0, 1, 3, 2).reshape(b, s, f)


# speedup vs baseline: 1.0015x; 1.0015x over previous
import jax
import jax.numpy as jnp
from jax.experimental import pallas as pl
from jax.experimental.pallas import tpu as pltpu

B, S, F = 4096, 4096, 4
R, L = 128, 128
BB = 128


def _mask_copy_kernel(idx_ref, x_ref, o_ref):
    i = pl.program_id(0)
    idx = idx_ref[0, i % 8, :].reshape(BB, 1, 1)
    row = jax.lax.broadcasted_iota(jnp.int32, (BB, R, 1), 1)
    lane = jax.lax.broadcasted_iota(jnp.int32, (BB, 1, L), 2)
    hit = ((row >> 2) == (idx >> 7)) & (lane == (idx & 127))
    o_ref[...] = jnp.where(hit, 0.0, x_ref[...])


def kernel(x, idx):
    b, s, f = x.shape
    v = x.reshape(b, s // L, L, f).transpose(0, 1, 3, 2).reshape(b, R, L)
    idx3 = idx.reshape(b // 1024, 8, 128)
    out = pl.pallas_call(
        _mask_copy_kernel,
        grid=(b // BB,),
        in_specs=[
            pl.BlockSpec((1, 8, 128), lambda i: (i // 8, 0, 0)),
            pl.BlockSpec((BB, R, L), lambda i: (i, 0, 0)),
        ],
        out_specs=pl.BlockSpec((BB, R, L), lambda i: (i, 0, 0)),
        out_shape=jax.ShapeDtypeStruct((b, R, L), x.dtype),
        compiler_params=pltpu.CompilerParams(vmem_limit_bytes=60000 * 1024),
    )(idx3, v)
    return out.reshape(b, s // L, f, L).transpose(0, 1, 3, 2).reshape(b, s, f)
